# baseline (speedup 1.0000x reference)
import functools

import jax
import jax.numpy as jnp
from jax import lax
from jax.experimental import pallas as pl
from jax.experimental.pallas import tpu as pltpu
from jax.experimental.pallas import tpu_sc as plsc

NUM_ROWS = 1000000
DIM = 64
NB = 4096
NT = 200
B_TOTAL = NB * NT

_info = plsc.get_sparse_core_info()
NC, NS = _info.num_cores, _info.num_subcores
NW = NC * NS
CHUNK = 256
N_BLK_TOTAL = B_TOTAL // CHUNK
N_BLK = N_BLK_TOTAL // NW
JJ = NB // CHUNK


@functools.partial(
    pl.kernel,
    out_type=jax.ShapeDtypeStruct((NT, DIM // 8, NB // 128, 8, 128), jnp.float32),
    mesh=plsc.VectorSubcoreMesh(core_axis_name="c", subcore_axis_name="s"),
    scratch_types=[
        pltpu.VMEM((2 * CHUNK,), jnp.int32),
        pltpu.VMEM((2, CHUNK, DIM), jnp.float32),
        pltpu.VMEM((2, DIM // 8, 2, 8, 128), jnp.float32),
        pltpu.SemaphoreType.DMA,
        pltpu.SemaphoreType.DMA,
        pltpu.SemaphoreType.DMA,
    ],
    compiler_params=pltpu.CompilerParams(
        use_tc_tiling_on_sc=False, needs_layout_passes=False),
)
def _gather_kernel(ids_hbm, w_hbm, out_hbm, idx_v, rows_v, stg_v, sem_i, sem_g, sem_s):
    wid = lax.axis_index("s") * NC + lax.axis_index("c")
    base = wid * N_BLK

    def idx_copy(n, b):
        n_c = jnp.minimum(n, N_BLK - 1)
        return pltpu.make_async_copy(
            ids_hbm.at[pl.ds((base + n_c) * CHUNK, CHUNK)],
            idx_v.at[pl.ds(b * CHUNK, CHUNK)], sem_i)

    def gather_copy(b):
        return pltpu.make_async_copy(
            w_hbm.at[idx_v.at[pl.ds(b * CHUNK, CHUNK)]], rows_v.at[b], sem_g)

    def store_copy(n, b):
        fb = base + n
        t = fb // JJ
        jj = fb % JJ
        return pltpu.make_async_copy(
            stg_v.at[b], out_hbm.at[t, :, pl.ds(2 * jj, 2)], sem_s)

    def transpose_block(b):
        rows = rows_v.at[b]
        for jj2 in range(2):
            for lg in range(8):
                l0 = 16 * lg
                kvec = lax.iota(jnp.int32, 16) + (128 * jj2 + l0)
                for d in range(DIM):
                    dvec = jnp.full((16,), d, jnp.int32)
                    v = plsc.load_gather(rows, [kvec, dvec])
                    stg_v[b, d // 8, jj2, d % 8, pl.ds(l0, 16)] = v

    idx_copy(0, 0).start()
    idx_copy(1, 1).start()
    idx_copy(0, 0).wait()
    gather_copy(0).start()

    @pl.loop(0, N_BLK, step=2)
    def _body(g):
        for b in range(2):
            n = g + b
            bn = (b + 1) % 2
            idx_copy(n + 1, bn).wait()

            @pl.when(n + 1 < N_BLK)
            def _():
                gather_copy(bn).start()

            gather_copy(b).wait()

            @pl.when(n >= 2)
            def _():
                store_copy(n - 2, b).wait()

            transpose_block(b)
            store_copy(n, b).start()
            idx_copy(n + 2, b).start()

    store_copy(N_BLK - 2, 0).wait()
    store_copy(N_BLK - 1, 1).wait()
    idx_copy(0, 0).wait()


def kernel(token_ids, weight):
    ids1 = token_ids.T.reshape(-1).astype(jnp.int32)
    out5 = _gather_kernel(ids1, weight)
    return out5.transpose(2, 4, 0, 1, 3).reshape(NB, NT, DIM)


# speedup vs baseline: 1.8324x; 1.8324x over previous
import functools

import jax
import jax.numpy as jnp
from jax import lax
from jax.experimental import pallas as pl
from jax.experimental.pallas import tpu as pltpu
from jax.experimental.pallas import tpu_sc as plsc

NUM_ROWS = 1000000
DIM = 64
NB = 4096
NT = 200
B_TOTAL = NB * NT

_info = plsc.get_sparse_core_info()
NC, NS = _info.num_cores, _info.num_subcores
NW = NC * NS
CHUNK = 256
N_BLK_TOTAL = B_TOTAL // CHUNK
N_BLK = N_BLK_TOTAL // NW
JJ = NB // CHUNK


@functools.partial(
    pl.kernel,
    out_type=jax.ShapeDtypeStruct((NT, DIM // 8, NB // 128, 8, 128), jnp.float32),
    mesh=plsc.VectorSubcoreMesh(core_axis_name="c", subcore_axis_name="s"),
    scratch_types=[
        pltpu.VMEM((2 * CHUNK,), jnp.int32),
        pltpu.VMEM((2, CHUNK, DIM), jnp.float32),
        pltpu.VMEM((2, 2, DIM, 130), jnp.float32),
        pltpu.SemaphoreType.DMA,
        pltpu.SemaphoreType.DMA,
        pltpu.SemaphoreType.DMA,
    ],
    compiler_params=pltpu.CompilerParams(
        use_tc_tiling_on_sc=False, needs_layout_passes=False),
)
def _gather_kernel(ids_hbm, w_hbm, out_hbm, idx_v, rows_v, stg_v, sem_i, sem_g, sem_s):
    wid = lax.axis_index("s") * NC + lax.axis_index("c")
    base = wid * N_BLK

    def idx_copy(n, b):
        n_c = jnp.minimum(n, N_BLK - 1)
        return pltpu.make_async_copy(
            ids_hbm.at[pl.ds((base + n_c) * CHUNK, CHUNK)],
            idx_v.at[pl.ds(b * CHUNK, CHUNK)], sem_i)

    def gather_copy(b):
        return pltpu.make_async_copy(
            w_hbm.at[idx_v.at[pl.ds(b * CHUNK, CHUNK)]], rows_v.at[b], sem_g)

    def store_block(n, b):
        fb = base + n
        t = fb // JJ
        jj = fb % JJ
        for jj2 in range(2):
            src = stg_v.at[b, jj2]
            for i in range(DIM // 8):
                pltpu.async_copy(
                    src.at[pl.ds(8 * i, 8), pl.ds(0, 128)],
                    out_hbm.at[t, i, 2 * jj + jj2], sem_s)

    def store_wait():
        for _ in range(16):
            pltpu.make_async_copy(
                stg_v.at[0, 0].at[pl.ds(0, 8), pl.ds(0, 128)],
                out_hbm.at[0, 0, 0], sem_s).wait()

    def transpose_block(b):
        rows = rows_v.at[b]
        iota = lax.iota(jnp.int32, 16)

        @pl.loop(0, 16)
        def _diag(j):
            rot = lax.bitwise_and(iota + j, 15)
            for jj2 in range(2):
                stg2 = stg_v.at[b, jj2]
                for k0 in range(0, 128, 16):
                    kvec = iota + (128 * jj2 + k0)
                    lvec = iota + k0
                    for d0 in range(0, DIM, 16):
                        dvec = rot + d0
                        v = plsc.load_gather(rows, [kvec, dvec])
                        plsc.store_scatter(stg2, [dvec, lvec], v)

    idx_copy(0, 0).start()
    idx_copy(1, 1).start()
    idx_copy(0, 0).wait()
    gather_copy(0).start()

    @pl.loop(0, N_BLK, step=2)
    def _body(g):
        for b in range(2):
            n = g + b
            bn = (b + 1) % 2
            idx_copy(n + 1, bn).wait()

            @pl.when(n + 1 < N_BLK)
            def _():
                gather_copy(bn).start()

            gather_copy(b).wait()

            @pl.when(n >= 2)
            def _():
                store_wait()

            transpose_block(b)
            store_block(n, b)
            idx_copy(n + 2, b).start()

    store_wait()
    store_wait()
    idx_copy(0, 0).wait()


def kernel(token_ids, weight):
    ids1 = token_ids.T.reshape(-1).astype(jnp.int32)
    out5 = _gather_kernel(ids1, weight)
    return out5.transpose(2, 4, 0, 1, 3).reshape(NB, NT, DIM)


# speedup vs baseline: 1.8410x; 1.0047x over previous
import functools

import jax
import jax.numpy as jnp
from jax import lax
from jax.experimental import pallas as pl
from jax.experimental.pallas import tpu as pltpu
from jax.experimental.pallas import tpu_sc as plsc

NUM_ROWS = 1000000
DIM = 64
NB = 4096
NT = 200
B_TOTAL = NB * NT

_info = plsc.get_sparse_core_info()
NC, NS = _info.num_cores, _info.num_subcores
NW = NC * NS
CHUNK = 256
N_BLK_TOTAL = B_TOTAL // CHUNK
N_BLK = N_BLK_TOTAL // NW
JJ = NB // CHUNK


@functools.partial(
    pl.kernel,
    out_type=jax.ShapeDtypeStruct((NT, DIM // 8, NB // 128, 8, 128), jnp.float32),
    mesh=plsc.VectorSubcoreMesh(core_axis_name="c", subcore_axis_name="s"),
    scratch_types=[
        pltpu.VMEM((2 * CHUNK,), jnp.int32),
        pltpu.VMEM((2, CHUNK, DIM), jnp.float32),
        pltpu.VMEM((2, DIM // 8, 2, 8, 130), jnp.float32),
        pltpu.SemaphoreType.DMA,
        pltpu.SemaphoreType.DMA,
        pltpu.SemaphoreType.DMA,
    ],
    compiler_params=pltpu.CompilerParams(
        use_tc_tiling_on_sc=False, needs_layout_passes=False),
)
def _gather_kernel(ids_hbm, w_hbm, out_hbm, idx_v, rows_v, stg_v, sem_i, sem_g, sem_s):
    wid = lax.axis_index("s") * NC + lax.axis_index("c")
    base = wid * N_BLK

    def idx_copy(n, b):
        n_c = jnp.minimum(n, N_BLK - 1)
        return pltpu.make_async_copy(
            ids_hbm.at[pl.ds((base + n_c) * CHUNK, CHUNK)],
            idx_v.at[pl.ds(b * CHUNK, CHUNK)], sem_i)

    def gather_copy(b):
        return pltpu.make_async_copy(
            w_hbm.at[idx_v.at[pl.ds(b * CHUNK, CHUNK)]], rows_v.at[b], sem_g)

    def store_block(n, b):
        fb = base + n
        t = fb // JJ
        jj = fb % JJ
        pltpu.async_copy(
            stg_v.at[b].at[:, :, :, pl.ds(0, 128)],
            out_hbm.at[t, :, pl.ds(2 * jj, 2)], sem_s)

    def store_wait():
        pltpu.make_async_copy(
            stg_v.at[0].at[:, :, :, pl.ds(0, 128)],
            out_hbm.at[0, :, pl.ds(0, 2)], sem_s).wait()

    def transpose_block(b):
        rows = rows_v.at[b]
        stg = stg_v.at[b]
        iota = lax.iota(jnp.int32, 16)

        @pl.loop(0, 16)
        def _diag(j):
            rot = lax.bitwise_and(iota + j, 15)
            dvecs = [rot + d0 for d0 in range(0, DIM, 16)]
            ivecs = [lax.shift_right_logical(dv, 3) for dv in dvecs]
            svecs = [lax.bitwise_and(dv, 7) for dv in dvecs]
            for jj2 in range(2):
                jjv = jnp.full((16,), jj2, jnp.int32)
                for k0 in range(0, 128, 16):
                    kvec = iota + (128 * jj2 + k0)
                    lvec = iota + k0
                    for c in range(DIM // 16):
                        v = plsc.load_gather(rows, [kvec, dvecs[c]])
                        plsc.store_scatter(stg, [ivecs[c], jjv, svecs[c], lvec], v)

    idx_copy(0, 0).start()
    idx_copy(1, 1).start()
    idx_copy(0, 0).wait()
    gather_copy(0).start()

    @pl.loop(0, N_BLK, step=2)
    def _body(g):
        for b in range(2):
            n = g + b
            bn = (b + 1) % 2
            idx_copy(n + 1, bn).wait()

            @pl.when(n + 1 < N_BLK)
            def _():
                gather_copy(bn).start()

            gather_copy(b).wait()

            @pl.when(n >= 2)
            def _():
                store_wait()

            transpose_block(b)
            store_block(n, b)
            idx_copy(n + 2, b).start()

    store_wait()
    store_wait()
    idx_copy(0, 0).wait()


def kernel(token_ids, weight):
    ids1 = token_ids.T.reshape(-1).astype(jnp.int32)
    out5 = _gather_kernel(ids1, weight)
    return out5.transpose(2, 4, 0, 1, 3).reshape(NB, NT, DIM)


# speedup vs baseline: 1.9152x; 1.0403x over previous
import functools

import jax
import jax.numpy as jnp
from jax import lax
from jax.experimental import pallas as pl
from jax.experimental.pallas import tpu as pltpu
from jax.experimental.pallas import tpu_sc as plsc

NUM_ROWS = 1000000
DIM = 64
NB = 4096
NT = 200
B_TOTAL = NB * NT

_info = plsc.get_sparse_core_info()
NC, NS = _info.num_cores, _info.num_subcores
NW = NC * NS
CHUNK = 256
N_BLK_TOTAL = B_TOTAL // CHUNK
N_BLK = N_BLK_TOTAL // NW
JJ = NB // CHUNK


@functools.partial(
    pl.kernel,
    out_type=jax.ShapeDtypeStruct((NT, DIM // 8, NB // 128, 8, 128), jnp.float32),
    mesh=plsc.VectorSubcoreMesh(core_axis_name="c", subcore_axis_name="s"),
    scratch_types=[
        pltpu.VMEM((2 * CHUNK,), jnp.int32),
        pltpu.VMEM((2, CHUNK, DIM), jnp.float32),
        pltpu.VMEM((2, DIM // 8, 2, 8, 130), jnp.float32),
        pltpu.SemaphoreType.DMA,
        pltpu.SemaphoreType.DMA,
        pltpu.SemaphoreType.DMA,
    ],
    compiler_params=pltpu.CompilerParams(
        use_tc_tiling_on_sc=False, needs_layout_passes=False),
)
def _gather_kernel(ids_hbm, w_hbm, out_hbm, idx_v, rows_v, stg_v, sem_i, sem_g, sem_s):
    wid = lax.axis_index("s") * NC + lax.axis_index("c")
    base = wid * N_BLK

    def idx_copy(n, b):
        n_c = jnp.minimum(n, N_BLK - 1)
        return pltpu.make_async_copy(
            ids_hbm.at[pl.ds((base + n_c) * CHUNK, CHUNK)],
            idx_v.at[pl.ds(b * CHUNK, CHUNK)], sem_i)

    def gather_copy(b):
        return pltpu.make_async_copy(
            w_hbm.at[idx_v.at[pl.ds(b * CHUNK, CHUNK)]], rows_v.at[b], sem_g)

    def store_block(n, b):
        fb = base + n
        t = fb // JJ
        jj = fb % JJ
        pltpu.async_copy(
            stg_v.at[b].at[:, :, :, pl.ds(0, 128)],
            out_hbm.at[t, :, pl.ds(2 * jj, 2)], sem_s)

    def store_wait():
        pltpu.make_async_copy(
            stg_v.at[0].at[:, :, :, pl.ds(0, 128)],
            out_hbm.at[0, :, pl.ds(0, 2)], sem_s).wait()

    def transpose_block(b):
        rows = rows_v.at[b]
        stg = stg_v.at[b]
        iota = lax.iota(jnp.int32, 16)

        @pl.loop(0, 16, unroll=4)
        def _diag(j):
            rot = lax.bitwise_and(iota + j, 15)
            dvecs = [rot + d0 for d0 in range(0, DIM, 16)]
            ivecs = [lax.shift_right_logical(dv, 3) for dv in dvecs]
            svecs = [lax.bitwise_and(dv, 7) for dv in dvecs]
            for jj2 in range(2):
                jjv = jnp.full((16,), jj2, jnp.int32)
                for k0 in range(0, 128, 16):
                    kvec = iota + (128 * jj2 + k0)
                    lvec = iota + k0
                    for c in range(DIM // 16):
                        v = plsc.load_gather(rows, [kvec, dvecs[c]])
                        plsc.store_scatter(stg, [ivecs[c], jjv, svecs[c], lvec], v)

    idx_copy(0, 0).start()
    idx_copy(1, 1).start()
    idx_copy(0, 0).wait()
    gather_copy(0).start()

    @pl.loop(0, N_BLK, step=2)
    def _body(g):
        for b in range(2):
            n = g + b
            bn = (b + 1) % 2
            idx_copy(n + 1, bn).wait()

            @pl.when(n + 1 < N_BLK)
            def _():
                gather_copy(bn).start()

            gather_copy(b).wait()

            @pl.when(n >= 2)
            def _():
                store_wait()

            transpose_block(b)
            store_block(n, b)
            idx_copy(n + 2, b).start()

    store_wait()
    store_wait()
    idx_copy(0, 0).wait()


def kernel(token_ids, weight):
    ids1 = token_ids.T.reshape(-1).astype(jnp.int32)
    out5 = _gather_kernel(ids1, weight)
    return out5.transpose(2, 4, 0, 1, 3).reshape(NB, NT, DIM)
